# baseline (speedup 1.0000x reference)
import jax
import jax.numpy as jnp
from jax.experimental import pallas as pl
from jax.experimental.pallas import tpu as pltpu
from jax.sharding import Mesh, PartitionSpec as P


def _expert0_matmul_kernel(x_ref, w_ref, o_ref):
    o_ref[...] = jax.lax.dot_general(
        x_ref[...],
        w_ref[...],
        dimension_numbers=(((1,), (1,)), ((), ())),
        preferred_element_type=jnp.float32,
    )


def _matmul_pallas(x, w):
    m, k = x.shape
    n = w.shape[0]
    bm = 2048 if m % 2048 == 0 else m
    return pl.pallas_call(
        _expert0_matmul_kernel,
        grid=(m // bm,),
        in_specs=[
            pl.BlockSpec((bm, k), lambda i: (i, 0)),
            pl.BlockSpec((n, k), lambda i: (0, 0)),
        ],
        out_specs=pl.BlockSpec((bm, n), lambda i: (i, 0)),
        out_shape=jax.ShapeDtypeStruct((m, n), x.dtype),
        compiler_params=pltpu.CompilerParams(
            dimension_semantics=("parallel",),
        ),
    )(x, w)


def kernel(inputs, router_w, expert_ws):
    del router_w
    w0 = expert_ws[0]
    m = inputs.shape[0]
    devs = jax.devices()
    ndev = len(devs) if m % max(len(devs), 1) == 0 else 1
    if ndev <= 1:
        return _matmul_pallas(inputs, w0)
    mesh = Mesh(devs, ("d",))
    fn = jax.shard_map(
        _matmul_pallas,
        mesh=mesh,
        in_specs=(P("d", None), P(None, None)),
        out_specs=P("d", None),
        check_vma=False,
    )
    return fn(inputs, w0)


# speedup vs baseline: 9.2961x; 9.2961x over previous
import jax
import jax.numpy as jnp
from jax.experimental import pallas as pl
from jax.experimental.pallas import tpu as pltpu


def _expert0_matmul_kernel(x_ref, w_ref, o_ref):
    o_ref[...] = jax.lax.dot_general(
        x_ref[...],
        w_ref[...],
        dimension_numbers=(((1,), (1,)), ((), ())),
        preferred_element_type=jnp.float32,
    )


def kernel(inputs, router_w, expert_ws):
    del router_w
    x = jax.lax.optimization_barrier(inputs.astype(jnp.bfloat16))
    w0 = jax.lax.optimization_barrier(expert_ws[0].astype(jnp.bfloat16))
    m, k = inputs.shape
    n = w0.shape[0]
    bm = 2048
    return pl.pallas_call(
        _expert0_matmul_kernel,
        grid=(m // bm,),
        in_specs=[
            pl.BlockSpec((bm, k), lambda i: (i, 0)),
            pl.BlockSpec((n, k), lambda i: (0, 0)),
        ],
        out_specs=pl.BlockSpec((bm, n), lambda i: (i, 0)),
        out_shape=jax.ShapeDtypeStruct((m, n), inputs.dtype),
        compiler_params=pltpu.CompilerParams(
            dimension_semantics=("parallel",),
        ),
    )(x, w0)


# speedup vs baseline: 12.7102x; 1.3673x over previous
import jax
import jax.numpy as jnp
from jax.experimental import pallas as pl
from jax.experimental.pallas import tpu as pltpu


def _expert0_matmul_kernel(x_ref, w_ref, o_ref):
    o_ref[...] = jax.lax.dot_general(
        x_ref[...],
        w_ref[...],
        dimension_numbers=(((1,), (1,)), ((), ())),
        preferred_element_type=jnp.float32,
    )


def kernel(inputs, router_w, expert_ws):
    del router_w
    w0 = expert_ws[0]
    m, k = inputs.shape
    n = w0.shape[0]
    bm = 1024
    return pl.pallas_call(
        _expert0_matmul_kernel,
        grid=(m // bm,),
        in_specs=[
            pl.BlockSpec((bm, k), lambda i: (i, 0)),
            pl.BlockSpec((n, k), lambda i: (0, 0)),
        ],
        out_specs=pl.BlockSpec((bm, n), lambda i: (i, 0)),
        out_shape=jax.ShapeDtypeStruct((m, n), inputs.dtype),
        compiler_params=pltpu.CompilerParams(
            dimension_semantics=("parallel",),
        ),
    )(inputs, w0)
